# baseline (speedup 1.0000x reference)
import functools

import jax
import jax.numpy as jnp
from jax import lax
from jax.experimental import pallas as pl
from jax.experimental.pallas import tpu as pltpu
from jax.experimental.pallas import tpu_sc as plsc

B = 16384
C = 26
V = 100000
D = 32

NC = 2
NS = 16
NW = NC * NS
L = 16
CB = 4096

_mesh = plsc.VectorSubcoreMesh(core_axis_name="c", subcore_axis_name="s")


@functools.partial(
    pl.kernel,
    out_type=jax.ShapeDtypeStruct((C * D, B), jnp.float32),
    mesh=_mesh,
    compiler_params=pltpu.CompilerParams(needs_layout_passes=False),
    scratch_types=[
        pltpu.VMEM((V,), jnp.float32),
        pltpu.VMEM((CB,), jnp.int32),
        pltpu.VMEM((CB,), jnp.float32),
        pltpu.VMEM((CB,), jnp.float32),
        pltpu.VMEM((CB,), jnp.float32),
        pltpu.VMEM((CB,), jnp.float32),
        pltpu.VMEM((C * D,), jnp.float32),
        pltpu.SemaphoreType.DMA,
        pltpu.SemaphoreType.DMA,
        pltpu.SemaphoreType.DMA,
        pltpu.SemaphoreType.DMA,
        pltpu.SemaphoreType.DMA,
    ],
)
def _embed(idx_hbm, tab_hbm, col_hbm, out_hbm,
           vec_v, idx_v, o0, o1, o2, o3, col_v,
           sem_v, sem_w0, sem_w1, sem_w2, sem_w3):
    w = lax.axis_index("s") * NC + lax.axis_index("c")

    def vdma(r):
        return pltpu.make_async_copy(tab_hbm.at[r], vec_v, sem_v)

    def wdma(o_ref, sem, r, hh):
        return pltpu.make_async_copy(
            o_ref, out_hbm.at[r, pl.ds(hh * CB, CB)], sem)

    pltpu.sync_copy(col_hbm, col_v)
    vdma(w).start()
    wdma(o0, sem_w0, w, 0).start()
    wdma(o1, sem_w1, w, 1).start()
    wdma(o2, sem_w2, w, 2).start()
    wdma(o3, sem_w3, w, 3).start()

    def per_c(c, carry):
        row = c * D + w
        nrow = jnp.minimum(c + 1, C - 1) * D + w
        bias = plsc.load_gather(col_v, [jnp.full((L,), row, jnp.int32)])

        vdma(row).wait()
        obufs = (o0, o1, o2, o3)
        sems = (sem_w0, sem_w1, sem_w2, sem_w3)
        for k in range(B // CB):
            ob, sem = obufs[k], sems[k]
            pltpu.sync_copy(idx_hbm.at[c, pl.ds(k * CB, CB)], idx_v)
            wdma(ob, sem, row, k).wait()

            @plsc.parallel_loop(0, CB // L, unroll=8)
            def _g(i, ob=ob):
                ids = idx_v[pl.ds(i * L, L)]
                ob[pl.ds(i * L, L)] = plsc.load_gather(vec_v, [ids]) + bias

            if k == B // CB - 1:
                vdma(nrow).start()
            wdma(ob, sem, row, k).start()
        return carry

    lax.fori_loop(0, C, per_c, 0)
    last = (C - 1) * D + w
    vdma(last).wait()
    wdma(o0, sem_w0, last, 0).wait()
    wdma(o1, sem_w1, last, 1).wait()
    wdma(o2, sem_w2, last, 2).wait()
    wdma(o3, sem_w3, last, 3).wait()


def kernel(indices, tables, col_table):
    idx_t = indices.astype(jnp.int32).T
    tab_t = tables.transpose(0, 2, 1).reshape(C * D, V)
    out = _embed(idx_t, tab_t, col_table.reshape(C * D))
    return out.reshape(C, D, B).transpose(2, 0, 1)


# speedup vs baseline: 1.1968x; 1.1968x over previous
import functools

import jax
import jax.numpy as jnp
from jax import lax
from jax.experimental import pallas as pl
from jax.experimental.pallas import tpu as pltpu
from jax.experimental.pallas import tpu_sc as plsc

B = 16384
C = 26
V = 100000
D = 32

NC = 2
NS = 16
NW = NC * NS
L = 16
CB = 8192

_mesh = plsc.VectorSubcoreMesh(core_axis_name="c", subcore_axis_name="s")


@functools.partial(
    pl.kernel,
    out_type=jax.ShapeDtypeStruct((C * D, B), jnp.float32),
    mesh=_mesh,
    compiler_params=pltpu.CompilerParams(needs_layout_passes=False),
    scratch_types=[
        pltpu.VMEM((V,), jnp.float32),
        pltpu.VMEM((CB,), jnp.int32),
        pltpu.VMEM((B,), jnp.float32),
        pltpu.VMEM((C * D,), jnp.float32),
        pltpu.SemaphoreType.DMA,
        pltpu.SemaphoreType.DMA,
    ],
)
def _embed(idx_hbm, tab_hbm, col_hbm, out_hbm,
           vec_v, idx_v, o_v, col_v, sem_v, sem_w):
    w = lax.axis_index("s") * NC + lax.axis_index("c")

    def vdma(r):
        return pltpu.make_async_copy(tab_hbm.at[r], vec_v, sem_v)

    def wdma(r):
        return pltpu.make_async_copy(o_v, out_hbm.at[r], sem_w)

    pltpu.sync_copy(col_hbm, col_v)
    vdma(w).start()
    wdma(w).start()

    def per_c(c, carry):
        row = c * D + w
        nrow = jnp.minimum(c + 1, C - 1) * D + w
        bias = plsc.load_gather(col_v, [jnp.full((L,), row, jnp.int32)])

        pltpu.sync_copy(idx_hbm.at[c, pl.ds(0, CB)], idx_v)
        vdma(row).wait()
        wdma(row).wait()

        @plsc.parallel_loop(0, CB // L, unroll=8)
        def _g0(i):
            ids = idx_v[pl.ds(i * L, L)]
            o_v[pl.ds(i * L, L)] = plsc.load_gather(vec_v, [ids]) + bias

        pltpu.sync_copy(idx_hbm.at[c, pl.ds(CB, CB)], idx_v)

        @plsc.parallel_loop(0, CB // L, unroll=8)
        def _g1(i):
            ids = idx_v[pl.ds(i * L, L)]
            o_v[pl.ds(CB + i * L, L)] = plsc.load_gather(vec_v, [ids]) + bias

        vdma(nrow).start()
        wdma(row).start()
        return carry

    lax.fori_loop(0, C, per_c, 0)
    last = (C - 1) * D + w
    vdma(last).wait()
    wdma(last).wait()


def kernel(indices, tables, col_table):
    idx_t = indices.astype(jnp.int32).T
    tab_t = tables.transpose(0, 2, 1).reshape(C * D, V)
    out = _embed(idx_t, tab_t, col_table.reshape(C * D))
    return out.reshape(C, D, B).transpose(2, 0, 1)
